# Initial kernel scaffold; baseline (speedup 1.0000x reference)
import jax
import jax.numpy as jnp
from jax import lax
from jax.experimental import pallas as pl
from jax.experimental.pallas import tpu as pltpu
from jax.experimental.pallas import tpu_sc as plsc

N = 10000
D = 128
E = 320000
W2COL = D + 16
NP = 10240
NTILES = 32
CHUNK = 128
CPT = 79
EP = NTILES * CPT * CHUNK
RPT = NP // 16

_MESH = plsc.VectorSubcoreMesh(core_axis_name="c", subcore_axis_name="s")


def _deg_body(src_hbm, dst_hbm, zeros_hbm, ones_hbm, out_hbm,
              srcv, dstv, ones_v, acc_o, acc_i):
    c = lax.axis_index("c")
    s = lax.axis_index("s")
    t = c * 16 + s
    sl = pl.ds(s * RPT, RPT)
    pltpu.sync_copy(zeros_hbm, acc_o.at[sl])
    pltpu.sync_copy(zeros_hbm, acc_i.at[sl])
    pltpu.sync_copy(ones_hbm, ones_v)
    pltpu.sync_copy(src_hbm.at[t], srcv)
    pltpu.sync_copy(dst_hbm.at[t], dstv)
    plsc.subcore_barrier()

    def body(i, carry):
        pltpu.sync_copy(ones_v, acc_o.at[srcv.at[i]], add=True)
        pltpu.sync_copy(ones_v, acc_i.at[dstv.at[i]], add=True)
        return carry

    lax.fori_loop(0, CPT, body, 0)
    plsc.subcore_barrier()
    pltpu.sync_copy(acc_o.at[sl], out_hbm.at[0, c, sl])
    pltpu.sync_copy(acc_i.at[sl], out_hbm.at[1, c, sl])


_deg_call = pl.kernel(
    _deg_body,
    out_type=jax.ShapeDtypeStruct((2, 2, NP, 16), jnp.float32),
    mesh=_MESH,
    scratch_types=[
        pltpu.VMEM((CPT, CHUNK), jnp.int32),
        pltpu.VMEM((CPT, CHUNK), jnp.int32),
        pltpu.VMEM((CHUNK, 16), jnp.float32),
        pltpu.VMEM_SHARED((NP, 16), jnp.float32),
        pltpu.VMEM_SHARED((NP, 16), jnp.float32),
    ],
)


def _make_segsum(width):
    def body(table_hbm, src_hbm, dst_hbm, zeros_hbm, out_hbm,
             srcv, dstv, rows, acc, gsem):
        c = lax.axis_index("c")
        s = lax.axis_index("s")
        t = c * 16 + s
        sl = pl.ds(s * RPT, RPT)
        pltpu.sync_copy(zeros_hbm, acc.at[sl])
        pltpu.sync_copy(src_hbm.at[t], srcv)
        pltpu.sync_copy(dst_hbm.at[t], dstv)
        plsc.subcore_barrier()
        pltpu.async_copy(table_hbm.at[srcv.at[0]], rows.at[0], gsem)

        def body_fn(i, carry):
            slot = lax.rem(i, 2)
            nslot = lax.rem(i + 1, 2)

            @pl.when(i + 1 < CPT)
            def _():
                pltpu.async_copy(table_hbm.at[srcv.at[i + 1]], rows.at[nslot], gsem)

            pltpu.make_async_copy(table_hbm.at[srcv.at[i]], rows.at[slot], gsem).wait()
            pltpu.sync_copy(rows.at[slot], acc.at[dstv.at[i]], add=True)
            return carry

        lax.fori_loop(0, CPT, body_fn, 0)
        plsc.subcore_barrier()
        pltpu.sync_copy(acc.at[sl], out_hbm.at[c, sl])

    return pl.kernel(
        body,
        out_type=jax.ShapeDtypeStruct((2, NP, width), jnp.float32),
        mesh=_MESH,
        scratch_types=[
            pltpu.VMEM((CPT, CHUNK), jnp.int32),
            pltpu.VMEM((CPT, CHUNK), jnp.int32),
            pltpu.VMEM((2, CHUNK, width), jnp.float32),
            pltpu.VMEM_SHARED((NP, width), jnp.float32),
            pltpu.SemaphoreType.DMA,
        ],
    )


_seg128 = _make_segsum(D)
_seg144 = _make_segsum(W2COL)


BLK = 1024
NB = NP // BLK

_deg_spec = pl.BlockSpec((2, 2, BLK, 16), lambda i: (0, 0, i, 0))
_row_spec = pl.BlockSpec((BLK, D), lambda i: (i, 0))
_w_spec = pl.BlockSpec((D, D), lambda i: (0, 0))
_vec_spec = pl.BlockSpec((1, D), lambda i: (0, 0))
_one_spec = pl.BlockSpec((1, 1), lambda i: (0, 0))


def _tc_a(x_ref, w_ref, degs_ref, o_ref):
    dout = degs_ref[0, 0] + degs_ref[0, 1]
    n = lax.rsqrt(jnp.maximum(dout[:, 0:1], 1.0))
    o_ref[...] = jnp.dot(x_ref[...], w_ref[...],
                         preferred_element_type=jnp.float32) * n


_tc_call_a = pl.pallas_call(
    _tc_a,
    grid=(NB,),
    in_specs=[_row_spec, _w_spec, _deg_spec],
    out_specs=_row_spec,
    out_shape=jax.ShapeDtypeStruct((NP, D), jnp.float32),
)


def _tc_b(p1_ref, degs_ref, b1_ref, mask_ref, wl_ref, a_ref,
          h2_ref, e_ref, m_ref):
    i = pl.program_id(0)
    din = degs_ref[1, 0] + degs_ref[1, 1]
    n = lax.rsqrt(jnp.maximum(din[:, 0:1], 1.0))
    agg = p1_ref[0] + p1_ref[1]
    h = jnp.maximum(agg * n + b1_ref[...], 0.0)
    h2 = jnp.dot(h * mask_ref[...], wl_ref[...],
                 preferred_element_type=jnp.float32)
    h2_ref[...] = h2
    sc = jnp.sum(h2 * a_ref[...], axis=1, keepdims=True)
    e = jnp.where(sc > 0, sc, 0.01 * sc)
    e_ref[...] = jnp.broadcast_to(e, (BLK, D))

    @pl.when(i == 0)
    def _():
        m_ref[...] = jnp.full((1, 1), -jnp.inf, jnp.float32)

    m_ref[...] = jnp.maximum(m_ref[...], jnp.full((1, 1), jnp.max(e), jnp.float32))


_tc_call_b = pl.pallas_call(
    _tc_b,
    grid=(NB,),
    in_specs=[pl.BlockSpec((2, BLK, D), lambda i: (0, i, 0)),
              _deg_spec, _vec_spec, _vec_spec, _w_spec, _vec_spec],
    out_specs=[_row_spec, _row_spec, _one_spec],
    out_shape=[jax.ShapeDtypeStruct((NP, D), jnp.float32),
               jax.ShapeDtypeStruct((NP, D), jnp.float32),
               jax.ShapeDtypeStruct((1, 1), jnp.float32)],
)


def _tc_c(h2_ref, e_ref, m_ref, t2_ref):
    w = jnp.exp(e_ref[:, 0:1] - m_ref[...])
    t2_ref[:, 0:D] = h2_ref[...] * w
    t2_ref[:, D:W2COL] = jnp.broadcast_to(w, (BLK, 16))


_tc_call_c = pl.pallas_call(
    _tc_c,
    grid=(NB,),
    in_specs=[_row_spec, _row_spec, _one_spec],
    out_specs=pl.BlockSpec((BLK, W2COL), lambda i: (i, 0)),
    out_shape=jax.ShapeDtypeStruct((NP, W2COL), jnp.float32),
)


def _tc_d(p2_ref, degs_ref, w2_ref, t3_ref):
    num = p2_ref[0, :, 0:D] + p2_ref[1, :, 0:D]
    den = p2_ref[0, :, D:W2COL] + p2_ref[1, :, D:W2COL]
    h_att = jnp.maximum(num / jnp.maximum(den[:, 0:1], 1e-16), 0.0)
    dout = degs_ref[0, 0] + degs_ref[0, 1]
    n = lax.rsqrt(jnp.maximum(dout[:, 0:1], 1.0))
    t3_ref[...] = jnp.dot(h_att, w2_ref[...],
                          preferred_element_type=jnp.float32) * n


_tc_call_d = pl.pallas_call(
    _tc_d,
    grid=(NB,),
    in_specs=[pl.BlockSpec((2, BLK, W2COL), lambda i: (0, i, 0)),
              _deg_spec, _w_spec],
    out_specs=_row_spec,
    out_shape=jax.ShapeDtypeStruct((NP, D), jnp.float32),
)


def _tc_e(p3_ref, degs_ref, b2_ref, o_ref):
    din = degs_ref[1, 0] + degs_ref[1, 1]
    n = lax.rsqrt(jnp.maximum(din[:, 0:1], 1.0))
    o_ref[...] = jnp.maximum((p3_ref[0] + p3_ref[1]) * n + b2_ref[...], 0.0)


_tc_call_e = pl.pallas_call(
    _tc_e,
    grid=(NB,),
    in_specs=[pl.BlockSpec((2, BLK, D), lambda i: (0, i, 0)),
              _deg_spec, _vec_spec],
    out_specs=_row_spec,
    out_shape=jax.ShapeDtypeStruct((NP, D), jnp.float32),
)


def kernel(x, edge_index, mask, W1, b1, Wl, a, W2, b2):
    f32 = jnp.float32
    src = edge_index[0]
    dst = edge_index[1]
    padn = EP - E
    fill = (N + (jnp.arange(padn, dtype=jnp.int32) % (NP - N))).astype(jnp.int32)
    src_t = jnp.concatenate([src, fill]).reshape(NTILES, CPT, CHUNK)
    dst_t = jnp.concatenate([dst, fill]).reshape(NTILES, CPT, CHUNK)
    x_pad = jnp.concatenate([x, jnp.zeros((NP - N, D), f32)], axis=0)
    zeros16 = jnp.zeros((RPT, 16), f32)
    ones16 = jnp.ones((CHUNK, 16), f32)
    zeros128 = jnp.zeros((RPT, D), f32)
    zeros144 = jnp.zeros((RPT, W2COL), f32)
    b1r = b1.reshape(1, D)
    b2r = b2.reshape(1, D)
    maskr = mask.reshape(1, D)
    a_row = a.reshape(1, D)

    degs = _deg_call(src_t, dst_t, zeros16, ones16)
    t1 = _tc_call_a(x_pad, W1, degs)
    p1 = _seg128(t1, src_t, dst_t, zeros128)
    h2, e_bc, m = _tc_call_b(p1, degs, b1r, maskr, Wl, a_row)
    t2 = _tc_call_c(h2, e_bc, m)
    p2 = _seg144(t2, src_t, dst_t, zeros144)
    t3 = _tc_call_d(p2, degs, W2)
    p3 = _seg128(t3, src_t, dst_t, zeros128)
    out = _tc_call_e(p3, degs, b2r)
    return out[:N]


# speedup vs baseline: 16.6656x; 16.6656x over previous
import jax
import jax.numpy as jnp
from jax import lax
from jax.experimental import pallas as pl
from jax.experimental.pallas import tpu as pltpu
from jax.experimental.pallas import tpu_sc as plsc

N = 10000
D = 128
E = 320000
NP = 10240
NTILES = 32
CHUNK = 128
NPH = 2
CPP = 40
EP = NTILES * NPH * CPP * CHUNK
RPT = NP // 16

_MESH = plsc.VectorSubcoreMesh(core_axis_name="c", subcore_axis_name="s")


def _den_body(w_hbm, src_hbm, dst_hbm, zeros_hbm, out_hbm,
              srcv, dstv, vals, acc, gsem):
    c = lax.axis_index("c")
    s = lax.axis_index("s")
    t = c * 16 + s
    sl = pl.ds(s * RPT, RPT)
    pltpu.sync_copy(zeros_hbm, acc.at[sl])
    plsc.subcore_barrier()
    for ph in range(NPH):
        pltpu.sync_copy(src_hbm.at[t, ph], srcv)
        pltpu.sync_copy(dst_hbm.at[t, ph], dstv)
        pltpu.async_copy(w_hbm.at[srcv.at[0]], vals.at[0], gsem)

        def body(i, carry):
            slot = lax.rem(i, 2)
            nslot = lax.rem(i + 1, 2)

            @pl.when(i + 1 < CPP)
            def _():
                pltpu.async_copy(w_hbm.at[srcv.at[i + 1]], vals.at[nslot], gsem)

            pltpu.make_async_copy(w_hbm.at[srcv.at[i]], vals.at[slot], gsem).wait()
            pltpu.sync_copy(vals.at[slot], acc.at[dstv.at[i]], add=True)
            return carry

        lax.fori_loop(0, CPP, body, 0)
    plsc.subcore_barrier()
    base = pl.multiple_of(c * NP + s * RPT, 128)
    pltpu.sync_copy(acc.at[sl], out_hbm.at[pl.ds(base, RPT)])


_den_call = pl.kernel(
    _den_body,
    out_type=jax.ShapeDtypeStruct((2 * NP,), jnp.float32),
    mesh=_MESH,
    scratch_types=[
        pltpu.VMEM((CPP, CHUNK), jnp.int32),
        pltpu.VMEM((CPP, CHUNK), jnp.int32),
        pltpu.VMEM((2, CHUNK), jnp.float32),
        pltpu.VMEM_SHARED((NP,), jnp.float32),
        pltpu.SemaphoreType.DMA,
    ],
)


def _seg_body(table_hbm, src_hbm, dst_hbm, zeros_hbm, out_hbm,
              srcv, dstv, rows, acc, gsem):
    c = lax.axis_index("c")
    s = lax.axis_index("s")
    t = c * 16 + s
    sl = pl.ds(s * RPT, RPT)
    pltpu.sync_copy(zeros_hbm, acc.at[sl])
    plsc.subcore_barrier()
    for ph in range(NPH):
        pltpu.sync_copy(src_hbm.at[t, ph], srcv)
        pltpu.sync_copy(dst_hbm.at[t, ph], dstv)
        pltpu.async_copy(table_hbm.at[srcv.at[0]], rows.at[0], gsem)

        def body(i, carry):
            slot = lax.rem(i, 2)
            nslot = lax.rem(i + 1, 2)

            @pl.when(i + 1 < CPP)
            def _():
                pltpu.async_copy(table_hbm.at[srcv.at[i + 1]], rows.at[nslot], gsem)

            pltpu.make_async_copy(table_hbm.at[srcv.at[i]], rows.at[slot], gsem).wait()
            pltpu.sync_copy(rows.at[slot], acc.at[dstv.at[i]], add=True)
            return carry

        lax.fori_loop(0, CPP, body, 0)
    plsc.subcore_barrier()
    pltpu.sync_copy(acc.at[sl], out_hbm.at[c, sl])


_seg_call = pl.kernel(
    _seg_body,
    out_type=jax.ShapeDtypeStruct((2, NP, D), jnp.float32),
    mesh=_MESH,
    scratch_types=[
        pltpu.VMEM((CPP, CHUNK), jnp.int32),
        pltpu.VMEM((CPP, CHUNK), jnp.int32),
        pltpu.VMEM((2, CHUNK, D), jnp.float32),
        pltpu.VMEM_SHARED((NP, D), jnp.float32),
        pltpu.SemaphoreType.DMA,
    ],
)


BLK = 1024
NB = NP // BLK

_deg_spec = pl.BlockSpec((2, 2, BLK, 16), lambda i: (0, 0, i, 0))
_row_spec = pl.BlockSpec((BLK, D), lambda i: (i, 0))
_w_spec = pl.BlockSpec((D, D), lambda i: (0, 0))
_vec_spec = pl.BlockSpec((1, D), lambda i: (0, 0))
_one_spec = pl.BlockSpec((1, 1), lambda i: (0, 0))
_s16_spec = pl.BlockSpec((BLK, 16), lambda i: (i, 0))


def _tc_a(x_ref, w_ref, degs_ref, o_ref):
    dout = degs_ref[0, 0] + degs_ref[0, 1]
    n = lax.rsqrt(jnp.maximum(dout[:, 0:1], 1.0))
    o_ref[...] = jnp.dot(x_ref[...], w_ref[...],
                         preferred_element_type=jnp.float32) * n


_tc_call_a = pl.pallas_call(
    _tc_a,
    grid=(NB,),
    in_specs=[_row_spec, _w_spec, _deg_spec],
    out_specs=_row_spec,
    out_shape=jax.ShapeDtypeStruct((NP, D), jnp.float32),
)


def _tc_b(p1_ref, degs_ref, b1_ref, mask_ref, wl_ref, a_ref,
          h2_ref, e_ref, m_ref):
    i = pl.program_id(0)
    din = degs_ref[1, 0] + degs_ref[1, 1]
    n = lax.rsqrt(jnp.maximum(din[:, 0:1], 1.0))
    agg = p1_ref[0] + p1_ref[1]
    h = jnp.maximum(agg * n + b1_ref[...], 0.0)
    h2 = jnp.dot(h * mask_ref[...], wl_ref[...],
                 preferred_element_type=jnp.float32)
    h2_ref[...] = h2
    sc = jnp.sum(h2 * a_ref[...], axis=1, keepdims=True)
    e = jnp.where(sc > 0, sc, 0.01 * sc)
    e_ref[...] = jnp.broadcast_to(e, (BLK, 16))

    @pl.when(i == 0)
    def _():
        m_ref[...] = jnp.full((1, 1), -jnp.inf, jnp.float32)

    m_ref[...] = jnp.maximum(m_ref[...], jnp.full((1, 1), jnp.max(e), jnp.float32))


_tc_call_b = pl.pallas_call(
    _tc_b,
    grid=(NB,),
    in_specs=[pl.BlockSpec((2, BLK, D), lambda i: (0, i, 0)),
              _deg_spec, _vec_spec, _vec_spec, _w_spec, _vec_spec],
    out_specs=[_row_spec, _s16_spec, _one_spec],
    out_shape=[jax.ShapeDtypeStruct((NP, D), jnp.float32),
               jax.ShapeDtypeStruct((NP, 16), jnp.float32),
               jax.ShapeDtypeStruct((1, 1), jnp.float32)],
)


def _tc_c(h2_ref, e_ref, m_ref, t2_ref, w16_ref):
    w = jnp.exp(e_ref[:, 0:1] - m_ref[...])
    t2_ref[...] = h2_ref[...] * w
    w16_ref[...] = jnp.broadcast_to(w, (BLK, 16))


_tc_call_c = pl.pallas_call(
    _tc_c,
    grid=(NB,),
    in_specs=[_row_spec, _s16_spec, _one_spec],
    out_specs=[_row_spec, _s16_spec],
    out_shape=[jax.ShapeDtypeStruct((NP, D), jnp.float32),
               jax.ShapeDtypeStruct((NP, 16), jnp.float32)],
)


def _tc_d(p2_ref, den_ref, degs_ref, w2_ref, t3_ref):
    num = p2_ref[0] + p2_ref[1]
    den = den_ref[0] + den_ref[1]
    h_att = jnp.maximum(num / jnp.maximum(den[:, 0:1], 1e-16), 0.0)
    dout = degs_ref[0, 0] + degs_ref[0, 1]
    n = lax.rsqrt(jnp.maximum(dout[:, 0:1], 1.0))
    t3_ref[...] = jnp.dot(h_att, w2_ref[...],
                          preferred_element_type=jnp.float32) * n


_tc_call_d = pl.pallas_call(
    _tc_d,
    grid=(NB,),
    in_specs=[pl.BlockSpec((2, BLK, D), lambda i: (0, i, 0)),
              pl.BlockSpec((2, BLK, 16), lambda i: (0, i, 0)),
              _deg_spec, _w_spec],
    out_specs=_row_spec,
    out_shape=jax.ShapeDtypeStruct((NP, D), jnp.float32),
)


def _tc_e(p3_ref, degs_ref, b2_ref, o_ref):
    din = degs_ref[1, 0] + degs_ref[1, 1]
    n = lax.rsqrt(jnp.maximum(din[:, 0:1], 1.0))
    o_ref[...] = jnp.maximum((p3_ref[0] + p3_ref[1]) * n + b2_ref[...], 0.0)


_tc_call_e = pl.pallas_call(
    _tc_e,
    grid=(NB,),
    in_specs=[pl.BlockSpec((2, BLK, D), lambda i: (0, i, 0)),
              _deg_spec, _vec_spec],
    out_specs=_row_spec,
    out_shape=jax.ShapeDtypeStruct((NP, D), jnp.float32),
)


def kernel(x, edge_index, mask, W1, b1, Wl, a, W2, b2):
    f32 = jnp.float32
    src = edge_index[0]
    dst = edge_index[1]
    padn = EP - E
    fill = (N + (jnp.arange(padn, dtype=jnp.int32) % (NP - N))).astype(jnp.int32)
    src_t = jnp.concatenate([src, fill]).reshape(NTILES, NPH, CPP, CHUNK)
    dst_t = jnp.concatenate([dst, fill]).reshape(NTILES, NPH, CPP, CHUNK)
    src16_t = src_t * 16
    x_pad = jnp.concatenate([x, jnp.zeros((NP - N, D), f32)], axis=0)
    zeros1d = jnp.zeros((RPT,), f32)
    zerosrow = jnp.zeros((RPT, D), f32)
    b1r = b1.reshape(1, D)
    b2r = b2.reshape(1, D)
    maskr = mask.reshape(1, D)
    a_row = a.reshape(1, D)

    ones_flat = jnp.ones((NP * 16,), f32)
    dego_flat = _den_call(ones_flat, src16_t, src_t, zeros1d)
    degi_flat = _den_call(ones_flat, src16_t, dst_t, zeros1d)
    degs = jnp.broadcast_to(
        jnp.stack([dego_flat.reshape(2, NP), degi_flat.reshape(2, NP)],
                  axis=0)[:, :, :, None], (2, 2, NP, 16))
    t1 = _tc_call_a(x_pad, W1, degs)
    p1 = _seg_call(t1, src_t, dst_t, zerosrow)
    h2, e16, m = _tc_call_b(p1, degs, b1r, maskr, Wl, a_row)
    t2, w16 = _tc_call_c(h2, e16, m)
    p2 = _seg_call(t2, src_t, dst_t, zerosrow)
    den_flat = _den_call(w16.reshape(NP * 16), src16_t, dst_t, zeros1d)
    den = jnp.broadcast_to(den_flat.reshape(2, NP, 1), (2, NP, 16))
    t3 = _tc_call_d(p2, den, degs, W2)
    p3 = _seg_call(t3, src_t, dst_t, zerosrow)
    out = _tc_call_e(p3, degs, b2r)
    return out[:N]


# speedup vs baseline: 18.8804x; 1.1329x over previous
import jax
import jax.numpy as jnp
from jax import lax
from jax.experimental import pallas as pl
from jax.experimental.pallas import tpu as pltpu
from jax.experimental.pallas import tpu_sc as plsc

N = 10000
D = 128
E = 320000
NP = 10240
NTILES = 32
CHUNK = 128
NPH = 2
CPP = 40
EP = NTILES * NPH * CPP * CHUNK
RPT = NP // 16

_MESH = plsc.VectorSubcoreMesh(core_axis_name="c", subcore_axis_name="s")


def _den_body(w_hbm, src_hbm, dst_hbm, zeros_hbm, out_hbm,
              srcv, dstv, vals, acc, gsem):
    c = lax.axis_index("c")
    s = lax.axis_index("s")
    t = c * 16 + s
    sl = pl.ds(s * RPT, RPT)
    pltpu.sync_copy(zeros_hbm, acc.at[sl])
    plsc.subcore_barrier()
    for ph in range(NPH):
        pltpu.sync_copy(src_hbm.at[t, ph], srcv)
        pltpu.sync_copy(dst_hbm.at[t, ph], dstv)
        pltpu.async_copy(w_hbm.at[srcv.at[0]], vals.at[0], gsem)

        def body(i, carry):
            slot = lax.rem(i, 2)
            nslot = lax.rem(i + 1, 2)

            @pl.when(i + 1 < CPP)
            def _():
                pltpu.async_copy(w_hbm.at[srcv.at[i + 1]], vals.at[nslot], gsem)

            pltpu.make_async_copy(w_hbm.at[srcv.at[i]], vals.at[slot], gsem).wait()
            pltpu.sync_copy(vals.at[slot], acc.at[dstv.at[i]], add=True)
            return carry

        lax.fori_loop(0, CPP, body, 0)
    plsc.subcore_barrier()
    base = pl.multiple_of(c * NP + s * RPT, 128)
    pltpu.sync_copy(acc.at[sl], out_hbm.at[pl.ds(base, RPT)])


_den_call = pl.kernel(
    _den_body,
    out_type=jax.ShapeDtypeStruct((2 * NP,), jnp.float32),
    mesh=_MESH,
    scratch_types=[
        pltpu.VMEM((CPP, CHUNK), jnp.int32),
        pltpu.VMEM((CPP, CHUNK), jnp.int32),
        pltpu.VMEM((2, CHUNK), jnp.float32),
        pltpu.VMEM_SHARED((NP,), jnp.float32),
        pltpu.SemaphoreType.DMA,
    ],
)


def _deg2_body(src_hbm, dst_hbm, zeros_hbm, ones_hbm, out_hbm,
               idxv, ones_v, acc_o, acc_i):
    c = lax.axis_index("c")
    s = lax.axis_index("s")
    t = c * 16 + s
    sl = pl.ds(s * RPT, RPT)
    pltpu.sync_copy(zeros_hbm, acc_o.at[sl])
    pltpu.sync_copy(zeros_hbm, acc_i.at[sl])
    pltpu.sync_copy(ones_hbm, ones_v)
    plsc.subcore_barrier()
    for idx_hbm, acc in ((src_hbm, acc_o), (dst_hbm, acc_i)):
        for ph in range(NPH):
            pltpu.sync_copy(idx_hbm.at[t, ph], idxv)

            def body(i, carry):
                pltpu.sync_copy(ones_v, acc.at[idxv.at[i]], add=True)
                return carry

            lax.fori_loop(0, CPP, body, 0)
    plsc.subcore_barrier()
    base_o = pl.multiple_of(c * (2 * NP) + s * RPT, 128)
    pltpu.sync_copy(acc_o.at[sl], out_hbm.at[pl.ds(base_o, RPT)])
    base_i = pl.multiple_of(c * (2 * NP) + NP + s * RPT, 128)
    pltpu.sync_copy(acc_i.at[sl], out_hbm.at[pl.ds(base_i, RPT)])


_deg2_call = pl.kernel(
    _deg2_body,
    out_type=jax.ShapeDtypeStruct((2 * 2 * NP,), jnp.float32),
    mesh=_MESH,
    scratch_types=[
        pltpu.VMEM((CPP, CHUNK), jnp.int32),
        pltpu.VMEM((CHUNK,), jnp.float32),
        pltpu.VMEM_SHARED((NP,), jnp.float32),
        pltpu.VMEM_SHARED((NP,), jnp.float32),
    ],
)


def _seg_body(table_hbm, src_hbm, dst_hbm, zeros_hbm, out_hbm,
              srcv, dstv, rows, acc, gsem):
    c = lax.axis_index("c")
    s = lax.axis_index("s")
    t = c * 16 + s
    sl = pl.ds(s * RPT, RPT)
    pltpu.sync_copy(zeros_hbm, acc.at[sl])
    plsc.subcore_barrier()
    for ph in range(NPH):
        pltpu.sync_copy(src_hbm.at[t, ph], srcv)
        pltpu.sync_copy(dst_hbm.at[t, ph], dstv)
        pltpu.async_copy(table_hbm.at[srcv.at[0]], rows.at[0], gsem)

        def body(i, carry):
            slot = lax.rem(i, 2)
            nslot = lax.rem(i + 1, 2)

            @pl.when(i + 1 < CPP)
            def _():
                pltpu.async_copy(table_hbm.at[srcv.at[i + 1]], rows.at[nslot], gsem)

            pltpu.make_async_copy(table_hbm.at[srcv.at[i]], rows.at[slot], gsem).wait()
            pltpu.sync_copy(rows.at[slot], acc.at[dstv.at[i]], add=True)
            return carry

        lax.fori_loop(0, CPP, body, 0)
    plsc.subcore_barrier()
    pltpu.sync_copy(acc.at[sl], out_hbm.at[c, sl])


_seg_call = pl.kernel(
    _seg_body,
    out_type=jax.ShapeDtypeStruct((2, NP, D), jnp.float32),
    mesh=_MESH,
    scratch_types=[
        pltpu.VMEM((CPP, CHUNK), jnp.int32),
        pltpu.VMEM((CPP, CHUNK), jnp.int32),
        pltpu.VMEM((2, CHUNK, D), jnp.float32),
        pltpu.VMEM_SHARED((NP, D), jnp.float32),
        pltpu.SemaphoreType.DMA,
    ],
)


BLK = 1024
NB = NP // BLK

_deg_spec = pl.BlockSpec((2, 2, BLK, 16), lambda i: (0, 0, i, 0))
_row_spec = pl.BlockSpec((BLK, D), lambda i: (i, 0))
_w_spec = pl.BlockSpec((D, D), lambda i: (0, 0))
_vec_spec = pl.BlockSpec((1, D), lambda i: (0, 0))
_one_spec = pl.BlockSpec((1, 1), lambda i: (0, 0))
_s16_spec = pl.BlockSpec((BLK, 16), lambda i: (i, 0))


def _tc_a(x_ref, w_ref, degs_ref, o_ref):
    dout = degs_ref[0, 0] + degs_ref[0, 1]
    n = lax.rsqrt(jnp.maximum(dout[:, 0:1], 1.0))
    o_ref[...] = jnp.dot(x_ref[...], w_ref[...],
                         preferred_element_type=jnp.float32) * n


_tc_call_a = pl.pallas_call(
    _tc_a,
    grid=(NB,),
    in_specs=[_row_spec, _w_spec, _deg_spec],
    out_specs=_row_spec,
    out_shape=jax.ShapeDtypeStruct((NP, D), jnp.float32),
)


def _tc_b(p1_ref, degs_ref, b1_ref, mask_ref, wl_ref, a_ref,
          h2_ref, e_ref, m_ref):
    i = pl.program_id(0)
    din = degs_ref[1, 0] + degs_ref[1, 1]
    n = lax.rsqrt(jnp.maximum(din[:, 0:1], 1.0))
    agg = p1_ref[0] + p1_ref[1]
    h = jnp.maximum(agg * n + b1_ref[...], 0.0)
    h2 = jnp.dot(h * mask_ref[...], wl_ref[...],
                 preferred_element_type=jnp.float32)
    h2_ref[...] = h2
    sc = jnp.sum(h2 * a_ref[...], axis=1, keepdims=True)
    e = jnp.where(sc > 0, sc, 0.01 * sc)
    e_ref[...] = jnp.broadcast_to(e, (BLK, 16))

    @pl.when(i == 0)
    def _():
        m_ref[...] = jnp.full((1, 1), -jnp.inf, jnp.float32)

    m_ref[...] = jnp.maximum(m_ref[...], jnp.full((1, 1), jnp.max(e), jnp.float32))


_tc_call_b = pl.pallas_call(
    _tc_b,
    grid=(NB,),
    in_specs=[pl.BlockSpec((2, BLK, D), lambda i: (0, i, 0)),
              _deg_spec, _vec_spec, _vec_spec, _w_spec, _vec_spec],
    out_specs=[_row_spec, _s16_spec, _one_spec],
    out_shape=[jax.ShapeDtypeStruct((NP, D), jnp.float32),
               jax.ShapeDtypeStruct((NP, 16), jnp.float32),
               jax.ShapeDtypeStruct((1, 1), jnp.float32)],
)


def _tc_c(h2_ref, e_ref, m_ref, t2_ref, w16_ref):
    w = jnp.exp(e_ref[:, 0:1] - m_ref[...])
    t2_ref[...] = h2_ref[...] * w
    w16_ref[...] = jnp.broadcast_to(w, (BLK, 16))


_tc_call_c = pl.pallas_call(
    _tc_c,
    grid=(NB,),
    in_specs=[_row_spec, _s16_spec, _one_spec],
    out_specs=[_row_spec, _s16_spec],
    out_shape=[jax.ShapeDtypeStruct((NP, D), jnp.float32),
               jax.ShapeDtypeStruct((NP, 16), jnp.float32)],
)


def _tc_d(p2_ref, den_ref, degs_ref, w2_ref, t3_ref):
    num = p2_ref[0] + p2_ref[1]
    den = den_ref[0] + den_ref[1]
    h_att = jnp.maximum(num / jnp.maximum(den[:, 0:1], 1e-16), 0.0)
    dout = degs_ref[0, 0] + degs_ref[0, 1]
    n = lax.rsqrt(jnp.maximum(dout[:, 0:1], 1.0))
    t3_ref[...] = jnp.dot(h_att, w2_ref[...],
                          preferred_element_type=jnp.float32) * n


_tc_call_d = pl.pallas_call(
    _tc_d,
    grid=(NB,),
    in_specs=[pl.BlockSpec((2, BLK, D), lambda i: (0, i, 0)),
              pl.BlockSpec((2, BLK, 16), lambda i: (0, i, 0)),
              _deg_spec, _w_spec],
    out_specs=_row_spec,
    out_shape=jax.ShapeDtypeStruct((NP, D), jnp.float32),
)


def _tc_e(p3_ref, degs_ref, b2_ref, o_ref):
    din = degs_ref[1, 0] + degs_ref[1, 1]
    n = lax.rsqrt(jnp.maximum(din[:, 0:1], 1.0))
    o_ref[...] = jnp.maximum((p3_ref[0] + p3_ref[1]) * n + b2_ref[...], 0.0)


_tc_call_e = pl.pallas_call(
    _tc_e,
    grid=(NB,),
    in_specs=[pl.BlockSpec((2, BLK, D), lambda i: (0, i, 0)),
              _deg_spec, _vec_spec],
    out_specs=_row_spec,
    out_shape=jax.ShapeDtypeStruct((NP, D), jnp.float32),
)


def kernel(x, edge_index, mask, W1, b1, Wl, a, W2, b2):
    f32 = jnp.float32
    src = edge_index[0]
    dst = edge_index[1]
    padn = EP - E
    fill = (N + (jnp.arange(padn, dtype=jnp.int32) % (NP - N))).astype(jnp.int32)
    src_t = jnp.concatenate([src, fill]).reshape(NTILES, NPH, CPP, CHUNK)
    dst_t = jnp.concatenate([dst, fill]).reshape(NTILES, NPH, CPP, CHUNK)
    src16_t = src_t * 16
    x_pad = jnp.concatenate([x, jnp.zeros((NP - N, D), f32)], axis=0)
    zeros1d = jnp.zeros((RPT,), f32)
    zerosrow = jnp.zeros((RPT, D), f32)
    b1r = b1.reshape(1, D)
    b2r = b2.reshape(1, D)
    maskr = mask.reshape(1, D)
    a_row = a.reshape(1, D)

    ones128 = jnp.ones((CHUNK,), f32)
    d2_flat = _deg2_call(src_t, dst_t, zeros1d, ones128)
    degs = jnp.broadcast_to(
        d2_flat.reshape(2, 2, NP).transpose(1, 0, 2)[:, :, :, None],
        (2, 2, NP, 16))
    t1 = _tc_call_a(x_pad, W1, degs)
    p1 = _seg_call(t1, src_t, dst_t, zerosrow)
    h2, e16, m = _tc_call_b(p1, degs, b1r, maskr, Wl, a_row)
    t2, w16 = _tc_call_c(h2, e16, m)
    p2 = _seg_call(t2, src_t, dst_t, zerosrow)
    den_flat = _den_call(w16.reshape(NP * 16), src16_t, dst_t, zeros1d)
    den = jnp.broadcast_to(den_flat.reshape(2, NP, 1), (2, NP, 16))
    t3 = _tc_call_d(p2, den, degs, W2)
    p3 = _seg_call(t3, src_t, dst_t, zerosrow)
    out = _tc_call_e(p3, degs, b2r)
    return out[:N]


# speedup vs baseline: 19.7390x; 1.0455x over previous
import jax
import jax.numpy as jnp
from jax import lax
from jax.experimental import pallas as pl
from jax.experimental.pallas import tpu as pltpu
from jax.experimental.pallas import tpu_sc as plsc

N = 10000
D = 128
E = 320000
NP = 10240
NTILES = 32
CHUNK = 128
NPH = 2
CPP = 40
EP = NTILES * NPH * CPP * CHUNK
RPT = NP // 16

_MESH = plsc.VectorSubcoreMesh(core_axis_name="c", subcore_axis_name="s")


RING = 8
PREF = 4


def _den_body(w_hbm, src_hbm, dst_hbm, zeros_hbm, out_hbm,
              srcv, dstv, vals, acc, gsem, ssem):
    c = lax.axis_index("c")
    s = lax.axis_index("s")
    t = c * 16 + s
    sl = pl.ds(s * RPT, RPT)
    pltpu.sync_copy(zeros_hbm, acc.at[sl])
    plsc.subcore_barrier()
    for ph in range(NPH):
        pltpu.sync_copy(src_hbm.at[t, ph], srcv)
        pltpu.sync_copy(dst_hbm.at[t, ph], dstv)
        for j in range(PREF):
            pltpu.async_copy(w_hbm.at[srcv.at[j]], vals.at[j], gsem)

        def body(i, carry):
            slot = lax.rem(i, RING)

            @pl.when(i >= PREF)
            def _():
                pltpu.make_async_copy(
                    vals.at[lax.rem(i - PREF, RING)],
                    acc.at[dstv.at[i - PREF]], ssem).wait()

            @pl.when(i + PREF < CPP)
            def _():
                pltpu.async_copy(w_hbm.at[srcv.at[i + PREF]],
                                 vals.at[lax.rem(i + PREF, RING)], gsem)

            pltpu.make_async_copy(w_hbm.at[srcv.at[i]], vals.at[slot], gsem).wait()
            pltpu.async_copy(vals.at[slot], acc.at[dstv.at[i]], ssem, add=True)
            return carry

        lax.fori_loop(0, CPP, body, 0)
        for k in range(PREF):
            i = CPP - PREF + k
            pltpu.make_async_copy(vals.at[i % RING],
                                  acc.at[dstv.at[i]], ssem).wait()
    plsc.subcore_barrier()
    base = pl.multiple_of(c * NP + s * RPT, 128)
    pltpu.sync_copy(acc.at[sl], out_hbm.at[pl.ds(base, RPT)])


_den_call = pl.kernel(
    _den_body,
    out_type=jax.ShapeDtypeStruct((2 * NP,), jnp.float32),
    mesh=_MESH,
    scratch_types=[
        pltpu.VMEM((CPP, CHUNK), jnp.int32),
        pltpu.VMEM((CPP, CHUNK), jnp.int32),
        pltpu.VMEM((RING, CHUNK), jnp.float32),
        pltpu.VMEM_SHARED((NP,), jnp.float32),
        pltpu.SemaphoreType.DMA,
        pltpu.SemaphoreType.DMA,
    ],
)


def _deg2_body(src_hbm, dst_hbm, zeros_hbm, ones_hbm, out_hbm,
               idxv, ones_v, acc_o, acc_i):
    c = lax.axis_index("c")
    s = lax.axis_index("s")
    t = c * 16 + s
    sl = pl.ds(s * RPT, RPT)
    pltpu.sync_copy(zeros_hbm, acc_o.at[sl])
    pltpu.sync_copy(zeros_hbm, acc_i.at[sl])
    pltpu.sync_copy(ones_hbm, ones_v)
    plsc.subcore_barrier()
    for idx_hbm, acc in ((src_hbm, acc_o), (dst_hbm, acc_i)):
        for ph in range(NPH):
            pltpu.sync_copy(idx_hbm.at[t, ph], idxv)

            def body(i, carry):
                pltpu.sync_copy(ones_v, acc.at[idxv.at[i]], add=True)
                return carry

            lax.fori_loop(0, CPP, body, 0)
    plsc.subcore_barrier()
    base_o = pl.multiple_of(c * (2 * NP) + s * RPT, 128)
    pltpu.sync_copy(acc_o.at[sl], out_hbm.at[pl.ds(base_o, RPT)])
    base_i = pl.multiple_of(c * (2 * NP) + NP + s * RPT, 128)
    pltpu.sync_copy(acc_i.at[sl], out_hbm.at[pl.ds(base_i, RPT)])


_deg2_call = pl.kernel(
    _deg2_body,
    out_type=jax.ShapeDtypeStruct((2 * 2 * NP,), jnp.float32),
    mesh=_MESH,
    scratch_types=[
        pltpu.VMEM((CPP, CHUNK), jnp.int32),
        pltpu.VMEM((CHUNK,), jnp.float32),
        pltpu.VMEM_SHARED((NP,), jnp.float32),
        pltpu.VMEM_SHARED((NP,), jnp.float32),
    ],
)


def _seg_body(table_hbm, src_hbm, dst_hbm, zeros_hbm, out_hbm,
              srcv, dstv, rows, acc, gsem):
    c = lax.axis_index("c")
    s = lax.axis_index("s")
    t = c * 16 + s
    sl = pl.ds(s * RPT, RPT)
    pltpu.sync_copy(zeros_hbm, acc.at[sl])
    plsc.subcore_barrier()
    for ph in range(NPH):
        pltpu.sync_copy(src_hbm.at[t, ph], srcv)
        pltpu.sync_copy(dst_hbm.at[t, ph], dstv)
        pltpu.async_copy(table_hbm.at[srcv.at[0]], rows.at[0], gsem)

        def body(i, carry):
            slot = lax.rem(i, 2)
            nslot = lax.rem(i + 1, 2)

            @pl.when(i + 1 < CPP)
            def _():
                pltpu.async_copy(table_hbm.at[srcv.at[i + 1]], rows.at[nslot], gsem)

            pltpu.make_async_copy(table_hbm.at[srcv.at[i]], rows.at[slot], gsem).wait()
            pltpu.sync_copy(rows.at[slot], acc.at[dstv.at[i]], add=True)
            return carry

        lax.fori_loop(0, CPP, body, 0)
    plsc.subcore_barrier()
    pltpu.sync_copy(acc.at[sl], out_hbm.at[c, sl])


_seg_call = pl.kernel(
    _seg_body,
    out_type=jax.ShapeDtypeStruct((2, NP, D), jnp.float32),
    mesh=_MESH,
    scratch_types=[
        pltpu.VMEM((CPP, CHUNK), jnp.int32),
        pltpu.VMEM((CPP, CHUNK), jnp.int32),
        pltpu.VMEM((2, CHUNK, D), jnp.float32),
        pltpu.VMEM_SHARED((NP, D), jnp.float32),
        pltpu.SemaphoreType.DMA,
    ],
)


BLK = 2048
NB = NP // BLK
SL = 8

_deg_spec = pl.BlockSpec((2, 2, BLK, SL), lambda i: (0, 0, i, 0))
_row_spec = pl.BlockSpec((BLK, D), lambda i: (i, 0))
_w_spec = pl.BlockSpec((D, D), lambda i: (0, 0))
_vec_spec = pl.BlockSpec((1, D), lambda i: (0, 0))
_one_spec = pl.BlockSpec((1, 1), lambda i: (0, 0))
_s16_spec = pl.BlockSpec((BLK, SL), lambda i: (i, 0))


def _tc_a(x_ref, w_ref, degs_ref, o_ref):
    dout = degs_ref[0, 0] + degs_ref[0, 1]
    n = lax.rsqrt(jnp.maximum(dout[:, 0:1], 1.0))
    o_ref[...] = jnp.dot(x_ref[...], w_ref[...],
                         preferred_element_type=jnp.float32) * n


_tc_call_a = pl.pallas_call(
    _tc_a,
    grid=(NB,),
    in_specs=[_row_spec, _w_spec, _deg_spec],
    out_specs=_row_spec,
    out_shape=jax.ShapeDtypeStruct((NP, D), jnp.float32),
)


def _tc_b(p1_ref, degs_ref, b1_ref, mask_ref, wl_ref, a_ref,
          h2_ref, e_ref, m_ref):
    i = pl.program_id(0)
    din = degs_ref[1, 0] + degs_ref[1, 1]
    n = lax.rsqrt(jnp.maximum(din[:, 0:1], 1.0))
    agg = p1_ref[0] + p1_ref[1]
    h = jnp.maximum(agg * n + b1_ref[...], 0.0)
    h2 = jnp.dot(h * mask_ref[...], wl_ref[...],
                 preferred_element_type=jnp.float32)
    h2_ref[...] = h2
    sc = jnp.sum(h2 * a_ref[...], axis=1, keepdims=True)
    e = jnp.where(sc > 0, sc, 0.01 * sc)
    e_ref[...] = jnp.broadcast_to(e, (BLK, SL))

    @pl.when(i == 0)
    def _():
        m_ref[...] = jnp.full((1, 1), -jnp.inf, jnp.float32)

    m_ref[...] = jnp.maximum(m_ref[...], jnp.full((1, 1), jnp.max(e), jnp.float32))


_tc_call_b = pl.pallas_call(
    _tc_b,
    grid=(NB,),
    in_specs=[pl.BlockSpec((2, BLK, D), lambda i: (0, i, 0)),
              _deg_spec, _vec_spec, _vec_spec, _w_spec, _vec_spec],
    out_specs=[_row_spec, _s16_spec, _one_spec],
    out_shape=[jax.ShapeDtypeStruct((NP, D), jnp.float32),
               jax.ShapeDtypeStruct((NP, SL), jnp.float32),
               jax.ShapeDtypeStruct((1, 1), jnp.float32)],
)


def _tc_c(h2_ref, e_ref, m_ref, t2_ref, w16_ref):
    w = jnp.exp(e_ref[:, 0:1] - m_ref[...])
    t2_ref[...] = h2_ref[...] * w
    w16_ref[...] = jnp.broadcast_to(w, (BLK, SL))


_tc_call_c = pl.pallas_call(
    _tc_c,
    grid=(NB,),
    in_specs=[_row_spec, _s16_spec, _one_spec],
    out_specs=[_row_spec, _s16_spec],
    out_shape=[jax.ShapeDtypeStruct((NP, D), jnp.float32),
               jax.ShapeDtypeStruct((NP, SL), jnp.float32)],
)


def _tc_d(p2_ref, den_ref, degs_ref, w2_ref, t3_ref):
    num = p2_ref[0] + p2_ref[1]
    den = den_ref[0] + den_ref[1]
    h_att = jnp.maximum(num / jnp.maximum(den[:, 0:1], 1e-16), 0.0)
    dout = degs_ref[0, 0] + degs_ref[0, 1]
    n = lax.rsqrt(jnp.maximum(dout[:, 0:1], 1.0))
    t3_ref[...] = jnp.dot(h_att, w2_ref[...],
                          preferred_element_type=jnp.float32) * n


_tc_call_d = pl.pallas_call(
    _tc_d,
    grid=(NB,),
    in_specs=[pl.BlockSpec((2, BLK, D), lambda i: (0, i, 0)),
              pl.BlockSpec((2, BLK, SL), lambda i: (0, i, 0)),
              _deg_spec, _w_spec],
    out_specs=_row_spec,
    out_shape=jax.ShapeDtypeStruct((NP, D), jnp.float32),
)


def _tc_e(p3_ref, degs_ref, b2_ref, o_ref):
    din = degs_ref[1, 0] + degs_ref[1, 1]
    n = lax.rsqrt(jnp.maximum(din[:, 0:1], 1.0))
    o_ref[...] = jnp.maximum((p3_ref[0] + p3_ref[1]) * n + b2_ref[...], 0.0)


_tc_call_e = pl.pallas_call(
    _tc_e,
    grid=(NB,),
    in_specs=[pl.BlockSpec((2, BLK, D), lambda i: (0, i, 0)),
              _deg_spec, _vec_spec],
    out_specs=_row_spec,
    out_shape=jax.ShapeDtypeStruct((NP, D), jnp.float32),
)


def kernel(x, edge_index, mask, W1, b1, Wl, a, W2, b2):
    f32 = jnp.float32
    src = edge_index[0]
    dst = edge_index[1]
    padn = EP - E
    fill = (N + (jnp.arange(padn, dtype=jnp.int32) % (NP - N))).astype(jnp.int32)
    src_t = jnp.concatenate([src, fill]).reshape(NTILES, NPH, CPP, CHUNK)
    dst_t = jnp.concatenate([dst, fill]).reshape(NTILES, NPH, CPP, CHUNK)
    src8_t = src_t * SL
    x_pad = jnp.concatenate([x, jnp.zeros((NP - N, D), f32)], axis=0)
    zeros1d = jnp.zeros((RPT,), f32)
    zerosrow = jnp.zeros((RPT, D), f32)
    b1r = b1.reshape(1, D)
    b2r = b2.reshape(1, D)
    maskr = mask.reshape(1, D)
    a_row = a.reshape(1, D)

    ones128 = jnp.ones((CHUNK,), f32)
    d2_flat = _deg2_call(src_t, dst_t, zeros1d, ones128)
    degs = jnp.broadcast_to(
        d2_flat.reshape(2, 2, NP).transpose(1, 0, 2)[:, :, :, None],
        (2, 2, NP, SL))
    t1 = _tc_call_a(x_pad, W1, degs)
    p1 = _seg_call(t1, src_t, dst_t, zerosrow)
    h2, e16, m = _tc_call_b(p1, degs, b1r, maskr, Wl, a_row)
    t2, w16 = _tc_call_c(h2, e16, m)
    p2 = _seg_call(t2, src_t, dst_t, zerosrow)
    den_flat = _den_call(w16.reshape(NP * SL), src8_t, dst_t, zeros1d)
    den = jnp.broadcast_to(den_flat.reshape(2, NP, 1), (2, NP, SL))
    t3 = _tc_call_d(p2, den, degs, W2)
    p3 = _seg_call(t3, src_t, dst_t, zerosrow)
    out = _tc_call_e(p3, degs, b2r)
    return out[:N]


# speedup vs baseline: 19.7477x; 1.0004x over previous
import jax
import jax.numpy as jnp
from jax import lax
from jax.experimental import pallas as pl
from jax.experimental.pallas import tpu as pltpu
from jax.experimental.pallas import tpu_sc as plsc

N = 10000
D = 128
E = 320000
NP = 10240
NTILES = 32
CHUNK = 128
NPH = 2
CPP = 40
EP = NTILES * NPH * CPP * CHUNK
RPT = NP // 16

_MESH = plsc.VectorSubcoreMesh(core_axis_name="c", subcore_axis_name="s")


RING = 8
PREF = 4


def _den_body(w_hbm, src_hbm, dst_hbm, zeros_hbm, out_hbm,
              srcv, dstv, vals, acc, gsem, ssem):
    c = lax.axis_index("c")
    s = lax.axis_index("s")
    t = c * 16 + s
    sl = pl.ds(s * RPT, RPT)
    pltpu.sync_copy(zeros_hbm, acc.at[sl])
    plsc.subcore_barrier()
    for ph in range(NPH):
        pltpu.sync_copy(src_hbm.at[t, ph], srcv)
        pltpu.sync_copy(dst_hbm.at[t, ph], dstv)
        for j in range(PREF):
            pltpu.async_copy(w_hbm.at[srcv.at[j]], vals.at[j], gsem)

        def body(i, carry):
            slot = lax.rem(i, RING)

            @pl.when(i >= PREF)
            def _():
                pltpu.make_async_copy(
                    vals.at[lax.rem(i - PREF, RING)],
                    acc.at[dstv.at[i - PREF]], ssem).wait()

            @pl.when(i + PREF < CPP)
            def _():
                pltpu.async_copy(w_hbm.at[srcv.at[i + PREF]],
                                 vals.at[lax.rem(i + PREF, RING)], gsem)

            pltpu.make_async_copy(w_hbm.at[srcv.at[i]], vals.at[slot], gsem).wait()
            pltpu.async_copy(vals.at[slot], acc.at[dstv.at[i]], ssem, add=True)
            return carry

        lax.fori_loop(0, CPP, body, 0)
        for k in range(PREF):
            i = CPP - PREF + k
            pltpu.make_async_copy(vals.at[i % RING],
                                  acc.at[dstv.at[i]], ssem).wait()
    plsc.subcore_barrier()
    base = pl.multiple_of(c * NP + s * RPT, 128)
    pltpu.sync_copy(acc.at[sl], out_hbm.at[pl.ds(base, RPT)])


_den_call = pl.kernel(
    _den_body,
    out_type=jax.ShapeDtypeStruct((2 * NP,), jnp.float32),
    mesh=_MESH,
    scratch_types=[
        pltpu.VMEM((CPP, CHUNK), jnp.int32),
        pltpu.VMEM((CPP, CHUNK), jnp.int32),
        pltpu.VMEM((RING, CHUNK), jnp.float32),
        pltpu.VMEM_SHARED((NP,), jnp.float32),
        pltpu.SemaphoreType.DMA,
        pltpu.SemaphoreType.DMA,
    ],
)


def _deg2_body(src_hbm, dst_hbm, zeros_hbm, ones_hbm, out_hbm,
               idxv, ones_v, acc_o, acc_i):
    c = lax.axis_index("c")
    s = lax.axis_index("s")
    t = c * 16 + s
    sl = pl.ds(s * RPT, RPT)
    pltpu.sync_copy(zeros_hbm, acc_o.at[sl])
    pltpu.sync_copy(zeros_hbm, acc_i.at[sl])
    pltpu.sync_copy(ones_hbm, ones_v)
    plsc.subcore_barrier()
    for idx_hbm, acc in ((src_hbm, acc_o), (dst_hbm, acc_i)):
        for ph in range(NPH):
            pltpu.sync_copy(idx_hbm.at[t, ph], idxv)

            def body(i, carry):
                pltpu.sync_copy(ones_v, acc.at[idxv.at[i]], add=True)
                return carry

            lax.fori_loop(0, CPP, body, 0)
    plsc.subcore_barrier()
    base_o = pl.multiple_of(c * (2 * NP) + s * RPT, 128)
    pltpu.sync_copy(acc_o.at[sl], out_hbm.at[pl.ds(base_o, RPT)])
    base_i = pl.multiple_of(c * (2 * NP) + NP + s * RPT, 128)
    pltpu.sync_copy(acc_i.at[sl], out_hbm.at[pl.ds(base_i, RPT)])


_deg2_call = pl.kernel(
    _deg2_body,
    out_type=jax.ShapeDtypeStruct((2 * 2 * NP,), jnp.float32),
    mesh=_MESH,
    scratch_types=[
        pltpu.VMEM((CPP, CHUNK), jnp.int32),
        pltpu.VMEM((CHUNK,), jnp.float32),
        pltpu.VMEM_SHARED((NP,), jnp.float32),
        pltpu.VMEM_SHARED((NP,), jnp.float32),
    ],
)


def _seg_body(table_hbm, src_hbm, dst_hbm, zeros_hbm, out_hbm,
              srcv, dstv, rows, acc, gsem):
    c = lax.axis_index("c")
    s = lax.axis_index("s")
    t = c * 16 + s
    sl = pl.ds(s * RPT, RPT)
    pltpu.sync_copy(zeros_hbm, acc.at[sl])
    plsc.subcore_barrier()
    for ph in range(NPH):
        pltpu.sync_copy(src_hbm.at[t, ph], srcv)
        pltpu.sync_copy(dst_hbm.at[t, ph], dstv)
        pltpu.async_copy(table_hbm.at[srcv.at[0]], rows.at[0], gsem)

        def body(i, carry):
            slot = lax.rem(i, 2)
            nslot = lax.rem(i + 1, 2)

            @pl.when(i + 1 < CPP)
            def _():
                pltpu.async_copy(table_hbm.at[srcv.at[i + 1]], rows.at[nslot], gsem)

            pltpu.make_async_copy(table_hbm.at[srcv.at[i]], rows.at[slot], gsem).wait()
            pltpu.sync_copy(rows.at[slot], acc.at[dstv.at[i]], add=True)
            return carry

        lax.fori_loop(0, CPP, body, 0)
    plsc.subcore_barrier()
    pltpu.sync_copy(acc.at[sl], out_hbm.at[c, sl])


_seg_call = pl.kernel(
    _seg_body,
    out_type=jax.ShapeDtypeStruct((2, NP, D), jnp.float32),
    mesh=_MESH,
    scratch_types=[
        pltpu.VMEM((CPP, CHUNK), jnp.int32),
        pltpu.VMEM((CPP, CHUNK), jnp.int32),
        pltpu.VMEM((2, CHUNK, D), jnp.float32),
        pltpu.VMEM_SHARED((NP, D), jnp.float32),
        pltpu.SemaphoreType.DMA,
    ],
)


BLK = 2048
NB = NP // BLK
SL = 8

_deg_spec = pl.BlockSpec((2, 2, BLK, SL), lambda i: (0, 0, i, 0))
_row_spec = pl.BlockSpec((BLK, D), lambda i: (i, 0))
_w_spec = pl.BlockSpec((D, D), lambda i: (0, 0))
_vec_spec = pl.BlockSpec((1, D), lambda i: (0, 0))
_one_spec = pl.BlockSpec((1, 1), lambda i: (0, 0))
_s16_spec = pl.BlockSpec((BLK, SL), lambda i: (i, 0))


def _tc_a0(x_ref, w_ref, o_ref):
    o_ref[...] = jnp.dot(x_ref[...], w_ref[...],
                         preferred_element_type=jnp.float32)


_tc_call_a0 = pl.pallas_call(
    _tc_a0,
    grid=(NB,),
    in_specs=[_row_spec, _w_spec],
    out_specs=_row_spec,
    out_shape=jax.ShapeDtypeStruct((NP, D), jnp.float32),
)


def _tc_a1(y_ref, degs_ref, o_ref):
    dout = degs_ref[0, 0] + degs_ref[0, 1]
    n = lax.rsqrt(jnp.maximum(dout[:, 0:1], 1.0))
    o_ref[...] = y_ref[...] * n


_tc_call_a1 = pl.pallas_call(
    _tc_a1,
    grid=(NB,),
    in_specs=[_row_spec, _deg_spec],
    out_specs=_row_spec,
    out_shape=jax.ShapeDtypeStruct((NP, D), jnp.float32),
)


def _tc_b(p1_ref, degs_ref, b1_ref, mask_ref, wl_ref, a_ref,
          h2_ref, e_ref, m_ref):
    i = pl.program_id(0)
    din = degs_ref[1, 0] + degs_ref[1, 1]
    n = lax.rsqrt(jnp.maximum(din[:, 0:1], 1.0))
    agg = p1_ref[0] + p1_ref[1]
    h = jnp.maximum(agg * n + b1_ref[...], 0.0)
    h2 = jnp.dot(h * mask_ref[...], wl_ref[...],
                 preferred_element_type=jnp.float32)
    h2_ref[...] = h2
    sc = jnp.sum(h2 * a_ref[...], axis=1, keepdims=True)
    e = jnp.where(sc > 0, sc, 0.01 * sc)
    e_ref[...] = jnp.broadcast_to(e, (BLK, SL))

    @pl.when(i == 0)
    def _():
        m_ref[...] = jnp.full((1, 1), -jnp.inf, jnp.float32)

    m_ref[...] = jnp.maximum(m_ref[...], jnp.full((1, 1), jnp.max(e), jnp.float32))


_tc_call_b = pl.pallas_call(
    _tc_b,
    grid=(NB,),
    in_specs=[pl.BlockSpec((2, BLK, D), lambda i: (0, i, 0)),
              _deg_spec, _vec_spec, _vec_spec, _w_spec, _vec_spec],
    out_specs=[_row_spec, _s16_spec, _one_spec],
    out_shape=[jax.ShapeDtypeStruct((NP, D), jnp.float32),
               jax.ShapeDtypeStruct((NP, SL), jnp.float32),
               jax.ShapeDtypeStruct((1, 1), jnp.float32)],
)


def _tc_c(h2_ref, e_ref, m_ref, t2_ref, w16_ref):
    w = jnp.exp(e_ref[:, 0:1] - m_ref[...])
    t2_ref[...] = h2_ref[...] * w
    w16_ref[...] = jnp.broadcast_to(w, (BLK, SL))


_tc_call_c = pl.pallas_call(
    _tc_c,
    grid=(NB,),
    in_specs=[_row_spec, _s16_spec, _one_spec],
    out_specs=[_row_spec, _s16_spec],
    out_shape=[jax.ShapeDtypeStruct((NP, D), jnp.float32),
               jax.ShapeDtypeStruct((NP, SL), jnp.float32)],
)


def _tc_d(p2_ref, den_ref, degs_ref, w2_ref, t3_ref):
    num = p2_ref[0] + p2_ref[1]
    den = den_ref[0] + den_ref[1]
    h_att = jnp.maximum(num / jnp.maximum(den[:, 0:1], 1e-16), 0.0)
    dout = degs_ref[0, 0] + degs_ref[0, 1]
    n = lax.rsqrt(jnp.maximum(dout[:, 0:1], 1.0))
    t3_ref[...] = jnp.dot(h_att, w2_ref[...],
                          preferred_element_type=jnp.float32) * n


_tc_call_d = pl.pallas_call(
    _tc_d,
    grid=(NB,),
    in_specs=[pl.BlockSpec((2, BLK, D), lambda i: (0, i, 0)),
              pl.BlockSpec((2, BLK, SL), lambda i: (0, i, 0)),
              _deg_spec, _w_spec],
    out_specs=_row_spec,
    out_shape=jax.ShapeDtypeStruct((NP, D), jnp.float32),
)


def _tc_e(p3_ref, degs_ref, b2_ref, o_ref):
    din = degs_ref[1, 0] + degs_ref[1, 1]
    n = lax.rsqrt(jnp.maximum(din[:, 0:1], 1.0))
    o_ref[...] = jnp.maximum((p3_ref[0] + p3_ref[1]) * n + b2_ref[...], 0.0)


_tc_call_e = pl.pallas_call(
    _tc_e,
    grid=(NB,),
    in_specs=[pl.BlockSpec((2, BLK, D), lambda i: (0, i, 0)),
              _deg_spec, _vec_spec],
    out_specs=_row_spec,
    out_shape=jax.ShapeDtypeStruct((N, D), jnp.float32),
)


def kernel(x, edge_index, mask, W1, b1, Wl, a, W2, b2):
    f32 = jnp.float32
    src = edge_index[0]
    dst = edge_index[1]
    padn = EP - E
    fill = (N + (jnp.arange(padn, dtype=jnp.int32) % (NP - N))).astype(jnp.int32)
    src_t = jnp.concatenate([src, fill]).reshape(NTILES, NPH, CPP, CHUNK)
    dst_t = jnp.concatenate([dst, fill]).reshape(NTILES, NPH, CPP, CHUNK)
    src8_t = src_t * SL
    x_pad = jnp.concatenate([x, jnp.zeros((NP - N, D), f32)], axis=0)
    zeros1d = jnp.zeros((RPT,), f32)
    zerosrow = jnp.zeros((RPT, D), f32)
    b1r = b1.reshape(1, D)
    b2r = b2.reshape(1, D)
    maskr = mask.reshape(1, D)
    a_row = a.reshape(1, D)

    ones128 = jnp.ones((CHUNK,), f32)
    d2_flat = _deg2_call(src_t, dst_t, zeros1d, ones128)
    degs = jnp.broadcast_to(
        d2_flat.reshape(2, 2, NP).transpose(1, 0, 2)[:, :, :, None],
        (2, 2, NP, SL))
    y1 = _tc_call_a0(x_pad, W1)
    t1 = _tc_call_a1(y1, degs)
    p1 = _seg_call(t1, src_t, dst_t, zerosrow)
    h2, e16, m = _tc_call_b(p1, degs, b1r, maskr, Wl, a_row)
    t2, w16 = _tc_call_c(h2, e16, m)
    p2 = _seg_call(t2, src_t, dst_t, zerosrow)
    den_flat = _den_call(w16.reshape(NP * SL), src8_t, dst_t, zeros1d)
    den = jnp.broadcast_to(den_flat.reshape(2, NP, 1), (2, NP, SL))
    t3 = _tc_call_d(p2, den, degs, W2)
    p3 = _seg_call(t3, src_t, dst_t, zerosrow)
    return _tc_call_e(p3, degs, b2r)


# speedup vs baseline: 19.8786x; 1.0066x over previous
import jax
import jax.numpy as jnp
from jax import lax
from jax.experimental import pallas as pl
from jax.experimental.pallas import tpu as pltpu
from jax.experimental.pallas import tpu_sc as plsc

N = 10000
D = 128
E = 320000
NP = 10240
NTILES = 32
CHUNK = 128
NPH = 2
CPP = 40
EP = NTILES * NPH * CPP * CHUNK
RPT = NP // 16

_MESH = plsc.VectorSubcoreMesh(core_axis_name="c", subcore_axis_name="s")


RING = 8
PREF = 4


def _den_body(w_hbm, src_hbm, dst_hbm, zeros_hbm, out_hbm,
              srcv, dstv, vals, acc, gsem, ssem):
    c = lax.axis_index("c")
    s = lax.axis_index("s")
    t = c * 16 + s
    sl = pl.ds(s * RPT, RPT)
    pltpu.sync_copy(zeros_hbm, acc.at[sl])
    plsc.subcore_barrier()
    for ph in range(NPH):
        pltpu.sync_copy(src_hbm.at[t, ph], srcv)
        pltpu.sync_copy(dst_hbm.at[t, ph], dstv)
        for j in range(PREF):
            pltpu.async_copy(w_hbm.at[srcv.at[j]], vals.at[j], gsem)

        def body(i, carry):
            slot = lax.rem(i, RING)

            @pl.when(i >= PREF)
            def _():
                pltpu.make_async_copy(
                    vals.at[lax.rem(i - PREF, RING)],
                    acc.at[dstv.at[i - PREF]], ssem).wait()

            @pl.when(i + PREF < CPP)
            def _():
                pltpu.async_copy(w_hbm.at[srcv.at[i + PREF]],
                                 vals.at[lax.rem(i + PREF, RING)], gsem)

            pltpu.make_async_copy(w_hbm.at[srcv.at[i]], vals.at[slot], gsem).wait()
            pltpu.async_copy(vals.at[slot], acc.at[dstv.at[i]], ssem, add=True)
            return carry

        lax.fori_loop(0, CPP, body, 0)
        for k in range(PREF):
            i = CPP - PREF + k
            pltpu.make_async_copy(vals.at[i % RING],
                                  acc.at[dstv.at[i]], ssem).wait()
    plsc.subcore_barrier()
    base = pl.multiple_of(c * NP + s * RPT, 128)
    pltpu.sync_copy(acc.at[sl], out_hbm.at[pl.ds(base, RPT)])


_den_call = pl.kernel(
    _den_body,
    out_type=jax.ShapeDtypeStruct((2 * NP,), jnp.float32),
    mesh=_MESH,
    scratch_types=[
        pltpu.VMEM((CPP, CHUNK), jnp.int32),
        pltpu.VMEM((CPP, CHUNK), jnp.int32),
        pltpu.VMEM((RING, CHUNK), jnp.float32),
        pltpu.VMEM_SHARED((NP,), jnp.float32),
        pltpu.SemaphoreType.DMA,
        pltpu.SemaphoreType.DMA,
    ],
)


def _deg2_body(src_hbm, dst_hbm, zeros_hbm, ones_hbm, out_hbm,
               idxv, ones_v, acc_o, acc_i):
    c = lax.axis_index("c")
    s = lax.axis_index("s")
    t = c * 16 + s
    sl = pl.ds(s * RPT, RPT)
    pltpu.sync_copy(zeros_hbm, acc_o.at[sl])
    pltpu.sync_copy(zeros_hbm, acc_i.at[sl])
    pltpu.sync_copy(ones_hbm, ones_v)
    plsc.subcore_barrier()
    for idx_hbm, acc in ((src_hbm, acc_o), (dst_hbm, acc_i)):
        for ph in range(NPH):
            pltpu.sync_copy(idx_hbm.at[t, ph], idxv)

            def body(i, carry):
                pltpu.sync_copy(ones_v, acc.at[idxv.at[i]], add=True)
                return carry

            lax.fori_loop(0, CPP, body, 0)
    plsc.subcore_barrier()
    base_o = pl.multiple_of(c * (2 * NP) + s * RPT, 128)
    pltpu.sync_copy(acc_o.at[sl], out_hbm.at[pl.ds(base_o, RPT)])
    base_i = pl.multiple_of(c * (2 * NP) + NP + s * RPT, 128)
    pltpu.sync_copy(acc_i.at[sl], out_hbm.at[pl.ds(base_i, RPT)])


_deg2_call = pl.kernel(
    _deg2_body,
    out_type=jax.ShapeDtypeStruct((2 * 2 * NP,), jnp.float32),
    mesh=_MESH,
    scratch_types=[
        pltpu.VMEM((CPP, CHUNK), jnp.int32),
        pltpu.VMEM((CHUNK,), jnp.float32),
        pltpu.VMEM_SHARED((NP,), jnp.float32),
        pltpu.VMEM_SHARED((NP,), jnp.float32),
    ],
)


def _seg_body(table_hbm, src_hbm, dst_hbm, zeros_hbm, out_hbm,
              srcv, dstv, rows, acc, gsem):
    c = lax.axis_index("c")
    s = lax.axis_index("s")
    t = c * 16 + s
    sl = pl.ds(s * RPT, RPT)
    pltpu.sync_copy(zeros_hbm, acc.at[sl])
    plsc.subcore_barrier()
    for ph in range(NPH):
        pltpu.sync_copy(src_hbm.at[t, ph], srcv)
        pltpu.sync_copy(dst_hbm.at[t, ph], dstv)
        pltpu.async_copy(table_hbm.at[srcv.at[0]], rows.at[0], gsem)

        def body(i, carry):
            slot = lax.rem(i, 2)
            nslot = lax.rem(i + 1, 2)

            @pl.when(i + 1 < CPP)
            def _():
                pltpu.async_copy(table_hbm.at[srcv.at[i + 1]], rows.at[nslot], gsem)

            pltpu.make_async_copy(table_hbm.at[srcv.at[i]], rows.at[slot], gsem).wait()
            pltpu.sync_copy(rows.at[slot], acc.at[dstv.at[i]], add=True)
            return carry

        lax.fori_loop(0, CPP, body, 0)
    plsc.subcore_barrier()
    pltpu.sync_copy(acc.at[sl], out_hbm.at[c, sl])


_seg_call = pl.kernel(
    _seg_body,
    out_type=jax.ShapeDtypeStruct((2, NP, D), jnp.float32),
    mesh=_MESH,
    scratch_types=[
        pltpu.VMEM((CPP, CHUNK), jnp.int32),
        pltpu.VMEM((CPP, CHUNK), jnp.int32),
        pltpu.VMEM((2, CHUNK, D), jnp.float32),
        pltpu.VMEM_SHARED((NP, D), jnp.float32),
        pltpu.SemaphoreType.DMA,
    ],
)

def _segden_body(table_hbm, w_hbm, src_hbm, dst_hbm, src8_hbm,
                 zeros_hbm, zeros1_hbm, out_hbm, den_hbm,
                 srcv, dstv, rows, vals, acc, accd, gsem, ssem):
    c = lax.axis_index("c")
    s = lax.axis_index("s")
    t = c * 16 + s
    sl = pl.ds(s * RPT, RPT)
    pltpu.sync_copy(zeros_hbm, acc.at[sl])
    pltpu.sync_copy(zeros1_hbm, accd.at[sl])
    plsc.subcore_barrier()
    for ph in range(NPH):
        pltpu.sync_copy(src_hbm.at[t, ph], srcv)
        pltpu.sync_copy(dst_hbm.at[t, ph], dstv)
        pltpu.async_copy(table_hbm.at[srcv.at[0]], rows.at[0], gsem)

        def body(i, carry):
            slot = lax.rem(i, 2)
            nslot = lax.rem(i + 1, 2)

            @pl.when(i + 1 < CPP)
            def _():
                pltpu.async_copy(table_hbm.at[srcv.at[i + 1]], rows.at[nslot], gsem)

            pltpu.make_async_copy(table_hbm.at[srcv.at[i]], rows.at[slot], gsem).wait()
            pltpu.sync_copy(rows.at[slot], acc.at[dstv.at[i]], add=True)
            return carry

        lax.fori_loop(0, CPP, body, 0)
    for ph in range(NPH):
        pltpu.sync_copy(src8_hbm.at[t, ph], srcv)
        pltpu.sync_copy(dst_hbm.at[t, ph], dstv)
        for j in range(PREF):
            pltpu.async_copy(w_hbm.at[srcv.at[j]], vals.at[j], gsem)

        def dbody(i, carry):
            slot = lax.rem(i, RING)

            @pl.when(i >= PREF)
            def _():
                pltpu.make_async_copy(
                    vals.at[lax.rem(i - PREF, RING)],
                    accd.at[dstv.at[i - PREF]], ssem).wait()

            @pl.when(i + PREF < CPP)
            def _():
                pltpu.async_copy(w_hbm.at[srcv.at[i + PREF]],
                                 vals.at[lax.rem(i + PREF, RING)], gsem)

            pltpu.make_async_copy(w_hbm.at[srcv.at[i]], vals.at[slot], gsem).wait()
            pltpu.async_copy(vals.at[slot], accd.at[dstv.at[i]], ssem, add=True)
            return carry

        lax.fori_loop(0, CPP, dbody, 0)
        for k in range(PREF):
            i = CPP - PREF + k
            pltpu.make_async_copy(vals.at[i % RING],
                                  accd.at[dstv.at[i]], ssem).wait()
    plsc.subcore_barrier()
    pltpu.sync_copy(acc.at[sl], out_hbm.at[c, sl])
    base = pl.multiple_of(c * NP + s * RPT, 128)
    pltpu.sync_copy(accd.at[sl], den_hbm.at[pl.ds(base, RPT)])


_segden_call = pl.kernel(
    _segden_body,
    out_type=[jax.ShapeDtypeStruct((2, NP, D), jnp.float32),
              jax.ShapeDtypeStruct((2 * NP,), jnp.float32)],
    mesh=_MESH,
    scratch_types=[
        pltpu.VMEM((CPP, CHUNK), jnp.int32),
        pltpu.VMEM((CPP, CHUNK), jnp.int32),
        pltpu.VMEM((2, CHUNK, D), jnp.float32),
        pltpu.VMEM((RING, CHUNK), jnp.float32),
        pltpu.VMEM_SHARED((NP, D), jnp.float32),
        pltpu.VMEM_SHARED((NP,), jnp.float32),
        pltpu.SemaphoreType.DMA,
        pltpu.SemaphoreType.DMA,
    ],
)


BLK = 2048
NB = NP // BLK
SL = 8

_deg_spec = pl.BlockSpec((2, 2, BLK, SL), lambda i: (0, 0, i, 0))
_row_spec = pl.BlockSpec((BLK, D), lambda i: (i, 0))
_w_spec = pl.BlockSpec((D, D), lambda i: (0, 0))
_vec_spec = pl.BlockSpec((1, D), lambda i: (0, 0))
_one_spec = pl.BlockSpec((1, 1), lambda i: (0, 0))
_s16_spec = pl.BlockSpec((BLK, SL), lambda i: (i, 0))


def _tc_a0(x_ref, w_ref, o_ref):
    o_ref[...] = jnp.dot(x_ref[...], w_ref[...],
                         preferred_element_type=jnp.float32)


_tc_call_a0 = pl.pallas_call(
    _tc_a0,
    grid=(NB,),
    in_specs=[_row_spec, _w_spec],
    out_specs=_row_spec,
    out_shape=jax.ShapeDtypeStruct((NP, D), jnp.float32),
)


def _tc_a1(y_ref, degs_ref, o_ref):
    dout = degs_ref[0, 0] + degs_ref[0, 1]
    n = lax.rsqrt(jnp.maximum(dout[:, 0:1], 1.0))
    o_ref[...] = y_ref[...] * n


_tc_call_a1 = pl.pallas_call(
    _tc_a1,
    grid=(NB,),
    in_specs=[_row_spec, _deg_spec],
    out_specs=_row_spec,
    out_shape=jax.ShapeDtypeStruct((NP, D), jnp.float32),
)


def _tc_b(p1_ref, degs_ref, b1_ref, mask_ref, wl_ref, a_ref,
          h2_ref, e_ref, m_ref):
    i = pl.program_id(0)
    din = degs_ref[1, 0] + degs_ref[1, 1]
    n = lax.rsqrt(jnp.maximum(din[:, 0:1], 1.0))
    agg = p1_ref[0] + p1_ref[1]
    h = jnp.maximum(agg * n + b1_ref[...], 0.0)
    h2 = jnp.dot(h * mask_ref[...], wl_ref[...],
                 preferred_element_type=jnp.float32)
    h2_ref[...] = h2
    sc = jnp.sum(h2 * a_ref[...], axis=1, keepdims=True)
    e = jnp.where(sc > 0, sc, 0.01 * sc)
    e_ref[...] = jnp.broadcast_to(e, (BLK, SL))

    @pl.when(i == 0)
    def _():
        m_ref[...] = jnp.full((1, 1), -jnp.inf, jnp.float32)

    m_ref[...] = jnp.maximum(m_ref[...], jnp.full((1, 1), jnp.max(e), jnp.float32))


_tc_call_b = pl.pallas_call(
    _tc_b,
    grid=(NB,),
    in_specs=[pl.BlockSpec((2, BLK, D), lambda i: (0, i, 0)),
              _deg_spec, _vec_spec, _vec_spec, _w_spec, _vec_spec],
    out_specs=[_row_spec, _s16_spec, _one_spec],
    out_shape=[jax.ShapeDtypeStruct((NP, D), jnp.float32),
               jax.ShapeDtypeStruct((NP, SL), jnp.float32),
               jax.ShapeDtypeStruct((1, 1), jnp.float32)],
)


def _tc_c(h2_ref, e_ref, m_ref, t2_ref, w16_ref):
    w = jnp.exp(e_ref[:, 0:1] - m_ref[...])
    t2_ref[...] = h2_ref[...] * w
    w16_ref[...] = jnp.broadcast_to(w, (BLK, SL))


_tc_call_c = pl.pallas_call(
    _tc_c,
    grid=(NB,),
    in_specs=[_row_spec, _s16_spec, _one_spec],
    out_specs=[_row_spec, _s16_spec],
    out_shape=[jax.ShapeDtypeStruct((NP, D), jnp.float32),
               jax.ShapeDtypeStruct((NP, SL), jnp.float32)],
)


def _tc_d(p2_ref, den_ref, degs_ref, w2_ref, t3_ref):
    num = p2_ref[0] + p2_ref[1]
    den = den_ref[0] + den_ref[1]
    h_att = jnp.maximum(num / jnp.maximum(den[:, 0:1], 1e-16), 0.0)
    dout = degs_ref[0, 0] + degs_ref[0, 1]
    n = lax.rsqrt(jnp.maximum(dout[:, 0:1], 1.0))
    t3_ref[...] = jnp.dot(h_att, w2_ref[...],
                          preferred_element_type=jnp.float32) * n


_tc_call_d = pl.pallas_call(
    _tc_d,
    grid=(NB,),
    in_specs=[pl.BlockSpec((2, BLK, D), lambda i: (0, i, 0)),
              pl.BlockSpec((2, BLK, SL), lambda i: (0, i, 0)),
              _deg_spec, _w_spec],
    out_specs=_row_spec,
    out_shape=jax.ShapeDtypeStruct((NP, D), jnp.float32),
)


def _tc_e(p3_ref, degs_ref, b2_ref, o_ref):
    din = degs_ref[1, 0] + degs_ref[1, 1]
    n = lax.rsqrt(jnp.maximum(din[:, 0:1], 1.0))
    o_ref[...] = jnp.maximum((p3_ref[0] + p3_ref[1]) * n + b2_ref[...], 0.0)


_tc_call_e = pl.pallas_call(
    _tc_e,
    grid=(NB,),
    in_specs=[pl.BlockSpec((2, BLK, D), lambda i: (0, i, 0)),
              _deg_spec, _vec_spec],
    out_specs=_row_spec,
    out_shape=jax.ShapeDtypeStruct((N, D), jnp.float32),
)


def kernel(x, edge_index, mask, W1, b1, Wl, a, W2, b2):
    f32 = jnp.float32
    src = edge_index[0]
    dst = edge_index[1]
    padn = EP - E
    fill = (N + (jnp.arange(padn, dtype=jnp.int32) % (NP - N))).astype(jnp.int32)
    src_t = jnp.concatenate([src, fill]).reshape(NTILES, NPH, CPP, CHUNK)
    dst_t = jnp.concatenate([dst, fill]).reshape(NTILES, NPH, CPP, CHUNK)
    src8_t = src_t * SL
    x_pad = jnp.concatenate([x, jnp.zeros((NP - N, D), f32)], axis=0)
    zeros1d = jnp.zeros((RPT,), f32)
    zerosrow = jnp.zeros((RPT, D), f32)
    b1r = b1.reshape(1, D)
    b2r = b2.reshape(1, D)
    maskr = mask.reshape(1, D)
    a_row = a.reshape(1, D)

    ones128 = jnp.ones((CHUNK,), f32)
    d2_flat = _deg2_call(src_t, dst_t, zeros1d, ones128)
    degs = jnp.broadcast_to(
        d2_flat.reshape(2, 2, NP).transpose(1, 0, 2)[:, :, :, None],
        (2, 2, NP, SL))
    y1 = _tc_call_a0(x_pad, W1)
    t1 = _tc_call_a1(y1, degs)
    p1 = _seg_call(t1, src_t, dst_t, zerosrow)
    h2, e16, m = _tc_call_b(p1, degs, b1r, maskr, Wl, a_row)
    t2, w16 = _tc_call_c(h2, e16, m)
    p2, den_flat = _segden_call(t2, w16.reshape(NP * SL), src_t, dst_t,
                                src8_t, zerosrow, zeros1d)
    den = jnp.broadcast_to(den_flat.reshape(2, NP, 1), (2, NP, SL))
    t3 = _tc_call_d(p2, den, degs, W2)
    p3 = _seg_call(t3, src_t, dst_t, zerosrow)
    return _tc_call_e(p3, degs, b2r)


# speedup vs baseline: 20.2418x; 1.0183x over previous
import jax
import jax.numpy as jnp
from jax import lax
from jax.experimental import pallas as pl
from jax.experimental.pallas import tpu as pltpu
from jax.experimental.pallas import tpu_sc as plsc

N = 10000
D = 128
E = 320000
NP = 10240
NTILES = 32
CHUNK = 128
NPH = 2
CPP = 40
EP = NTILES * NPH * CPP * CHUNK
RPT = NP // 16

_MESH = plsc.VectorSubcoreMesh(core_axis_name="c", subcore_axis_name="s")


RING = 8
PREF = 4


def _deg2_body(src_hbm, dst_hbm, zeros_hbm, ones_hbm, out_hbm,
               idxv, ones_v, acc_o, acc_i, ssem):
    c = lax.axis_index("c")
    s = lax.axis_index("s")
    t = c * 16 + s
    sl = pl.ds(s * RPT, RPT)
    pltpu.sync_copy(zeros_hbm, acc_o.at[sl])
    pltpu.sync_copy(zeros_hbm, acc_i.at[sl])
    pltpu.sync_copy(ones_hbm, ones_v)
    plsc.subcore_barrier()
    for idx_hbm, acc in ((src_hbm, acc_o), (dst_hbm, acc_i)):
        for ph in range(NPH):
            pltpu.sync_copy(idx_hbm.at[t, ph], idxv)

            def body(i, carry):
                pltpu.async_copy(ones_v, acc.at[idxv.at[i]], ssem, add=True)
                return carry

            lax.fori_loop(0, CPP, body, 0)

            def drain(i, carry):
                pltpu.make_async_copy(ones_v, acc.at[idxv.at[0]], ssem).wait()
                return carry

            lax.fori_loop(0, CPP, drain, 0)
    plsc.subcore_barrier()
    base_o = pl.multiple_of(c * (2 * NP) + s * RPT, 128)
    pltpu.sync_copy(acc_o.at[sl], out_hbm.at[pl.ds(base_o, RPT)])
    base_i = pl.multiple_of(c * (2 * NP) + NP + s * RPT, 128)
    pltpu.sync_copy(acc_i.at[sl], out_hbm.at[pl.ds(base_i, RPT)])


_deg2_call = pl.kernel(
    _deg2_body,
    out_type=jax.ShapeDtypeStruct((2 * 2 * NP,), jnp.float32),
    mesh=_MESH,
    scratch_types=[
        pltpu.VMEM((CPP, CHUNK), jnp.int32),
        pltpu.VMEM((CHUNK,), jnp.float32),
        pltpu.VMEM_SHARED((NP,), jnp.float32),
        pltpu.VMEM_SHARED((NP,), jnp.float32),
        pltpu.SemaphoreType.DMA,
    ],
)


def _seg_body(table_hbm, src_hbm, dst_hbm, zeros_hbm, out_hbm,
              srcv, dstv, rows, acc, gsem):
    c = lax.axis_index("c")
    s = lax.axis_index("s")
    t = c * 16 + s
    sl = pl.ds(s * RPT, RPT)
    pltpu.sync_copy(zeros_hbm, acc.at[sl])
    plsc.subcore_barrier()
    for ph in range(NPH):
        pltpu.sync_copy(src_hbm.at[t, ph], srcv)
        pltpu.sync_copy(dst_hbm.at[t, ph], dstv)
        pltpu.async_copy(table_hbm.at[srcv.at[0]], rows.at[0], gsem)

        def body(i, carry):
            slot = lax.rem(i, 2)
            nslot = lax.rem(i + 1, 2)

            @pl.when(i + 1 < CPP)
            def _():
                pltpu.async_copy(table_hbm.at[srcv.at[i + 1]], rows.at[nslot], gsem)

            pltpu.make_async_copy(table_hbm.at[srcv.at[i]], rows.at[slot], gsem).wait()
            pltpu.sync_copy(rows.at[slot], acc.at[dstv.at[i]], add=True)
            return carry

        lax.fori_loop(0, CPP, body, 0)
    plsc.subcore_barrier()
    pltpu.sync_copy(acc.at[sl], out_hbm.at[c, sl])


_seg_call = pl.kernel(
    _seg_body,
    out_type=jax.ShapeDtypeStruct((2, NP, D), jnp.float32),
    mesh=_MESH,
    scratch_types=[
        pltpu.VMEM((CPP, CHUNK), jnp.int32),
        pltpu.VMEM((CPP, CHUNK), jnp.int32),
        pltpu.VMEM((2, CHUNK, D), jnp.float32),
        pltpu.VMEM_SHARED((NP, D), jnp.float32),
        pltpu.SemaphoreType.DMA,
    ],
)

def _segden_body(table_hbm, w_hbm, src_hbm, dst_hbm, src8_hbm,
                 zeros_hbm, zeros1_hbm, out_hbm, den_hbm,
                 srcv, dstv, rows, vals, acc, accd, gsem, ssem):
    c = lax.axis_index("c")
    s = lax.axis_index("s")
    t = c * 16 + s
    sl = pl.ds(s * RPT, RPT)
    pltpu.sync_copy(zeros_hbm, acc.at[sl])
    pltpu.sync_copy(zeros1_hbm, accd.at[sl])
    plsc.subcore_barrier()
    for ph in range(NPH):
        pltpu.sync_copy(src_hbm.at[t, ph], srcv)
        pltpu.sync_copy(dst_hbm.at[t, ph], dstv)
        pltpu.async_copy(table_hbm.at[srcv.at[0]], rows.at[0], gsem)

        def body(i, carry):
            slot = lax.rem(i, 2)
            nslot = lax.rem(i + 1, 2)

            @pl.when(i + 1 < CPP)
            def _():
                pltpu.async_copy(table_hbm.at[srcv.at[i + 1]], rows.at[nslot], gsem)

            pltpu.make_async_copy(table_hbm.at[srcv.at[i]], rows.at[slot], gsem).wait()
            pltpu.sync_copy(rows.at[slot], acc.at[dstv.at[i]], add=True)
            return carry

        lax.fori_loop(0, CPP, body, 0)
    for ph in range(NPH):
        pltpu.sync_copy(src8_hbm.at[t, ph], srcv)
        pltpu.sync_copy(dst_hbm.at[t, ph], dstv)
        for j in range(PREF):
            pltpu.async_copy(w_hbm.at[srcv.at[j]], vals.at[j], gsem)

        def dbody(i, carry):
            slot = lax.rem(i, RING)

            @pl.when(i >= PREF)
            def _():
                pltpu.make_async_copy(
                    vals.at[lax.rem(i - PREF, RING)],
                    accd.at[dstv.at[i - PREF]], ssem).wait()

            @pl.when(i + PREF < CPP)
            def _():
                pltpu.async_copy(w_hbm.at[srcv.at[i + PREF]],
                                 vals.at[lax.rem(i + PREF, RING)], gsem)

            pltpu.make_async_copy(w_hbm.at[srcv.at[i]], vals.at[slot], gsem).wait()
            pltpu.async_copy(vals.at[slot], accd.at[dstv.at[i]], ssem, add=True)
            return carry

        lax.fori_loop(0, CPP, dbody, 0)
        for k in range(PREF):
            i = CPP - PREF + k
            pltpu.make_async_copy(vals.at[i % RING],
                                  accd.at[dstv.at[i]], ssem).wait()
    plsc.subcore_barrier()
    pltpu.sync_copy(acc.at[sl], out_hbm.at[c, sl])
    base = pl.multiple_of(c * NP + s * RPT, 128)
    pltpu.sync_copy(accd.at[sl], den_hbm.at[pl.ds(base, RPT)])


_segden_call = pl.kernel(
    _segden_body,
    out_type=[jax.ShapeDtypeStruct((2, NP, D), jnp.float32),
              jax.ShapeDtypeStruct((2 * NP,), jnp.float32)],
    mesh=_MESH,
    scratch_types=[
        pltpu.VMEM((CPP, CHUNK), jnp.int32),
        pltpu.VMEM((CPP, CHUNK), jnp.int32),
        pltpu.VMEM((2, CHUNK, D), jnp.float32),
        pltpu.VMEM((RING, CHUNK), jnp.float32),
        pltpu.VMEM_SHARED((NP, D), jnp.float32),
        pltpu.VMEM_SHARED((NP,), jnp.float32),
        pltpu.SemaphoreType.DMA,
        pltpu.SemaphoreType.DMA,
    ],
)


BLK = 2048
NB = NP // BLK
SL = 8

_deg_spec = pl.BlockSpec((2, 2, BLK, SL), lambda i: (0, 0, i, 0))
_row_spec = pl.BlockSpec((BLK, D), lambda i: (i, 0))
_w_spec = pl.BlockSpec((D, D), lambda i: (0, 0))
_vec_spec = pl.BlockSpec((1, D), lambda i: (0, 0))
_one_spec = pl.BlockSpec((1, 1), lambda i: (0, 0))
_s16_spec = pl.BlockSpec((BLK, SL), lambda i: (i, 0))


def _tc_a0(x_ref, w_ref, o_ref):
    o_ref[...] = jnp.dot(x_ref[...], w_ref[...],
                         preferred_element_type=jnp.float32)


_tc_call_a0 = pl.pallas_call(
    _tc_a0,
    grid=(NB,),
    in_specs=[_row_spec, _w_spec],
    out_specs=_row_spec,
    out_shape=jax.ShapeDtypeStruct((NP, D), jnp.float32),
)


def _tc_a1(y_ref, degs_ref, o_ref):
    dout = degs_ref[0, 0] + degs_ref[0, 1]
    n = lax.rsqrt(jnp.maximum(dout[:, 0:1], 1.0))
    o_ref[...] = y_ref[...] * n


_tc_call_a1 = pl.pallas_call(
    _tc_a1,
    grid=(NB,),
    in_specs=[_row_spec, _deg_spec],
    out_specs=_row_spec,
    out_shape=jax.ShapeDtypeStruct((NP, D), jnp.float32),
)


def _tc_b(p1_ref, degs_ref, b1_ref, mask_ref, wl_ref, a_ref,
          h2_ref, e_ref, m_ref):
    i = pl.program_id(0)
    din = degs_ref[1, 0] + degs_ref[1, 1]
    n = lax.rsqrt(jnp.maximum(din[:, 0:1], 1.0))
    agg = p1_ref[0] + p1_ref[1]
    h = jnp.maximum(agg * n + b1_ref[...], 0.0)
    h2 = jnp.dot(h * mask_ref[...], wl_ref[...],
                 preferred_element_type=jnp.float32)
    h2_ref[...] = h2
    sc = jnp.sum(h2 * a_ref[...], axis=1, keepdims=True)
    e = jnp.where(sc > 0, sc, 0.01 * sc)
    e_ref[...] = jnp.broadcast_to(e, (BLK, SL))

    @pl.when(i == 0)
    def _():
        m_ref[...] = jnp.full((1, 1), -jnp.inf, jnp.float32)

    m_ref[...] = jnp.maximum(m_ref[...], jnp.full((1, 1), jnp.max(e), jnp.float32))


_tc_call_b = pl.pallas_call(
    _tc_b,
    grid=(NB,),
    in_specs=[pl.BlockSpec((2, BLK, D), lambda i: (0, i, 0)),
              _deg_spec, _vec_spec, _vec_spec, _w_spec, _vec_spec],
    out_specs=[_row_spec, _s16_spec, _one_spec],
    out_shape=[jax.ShapeDtypeStruct((NP, D), jnp.float32),
               jax.ShapeDtypeStruct((NP, SL), jnp.float32),
               jax.ShapeDtypeStruct((1, 1), jnp.float32)],
)


def _tc_c(h2_ref, e_ref, m_ref, t2_ref, w16_ref):
    w = jnp.exp(e_ref[:, 0:1] - m_ref[...])
    t2_ref[...] = h2_ref[...] * w
    w16_ref[...] = jnp.broadcast_to(w, (BLK, SL))


_tc_call_c = pl.pallas_call(
    _tc_c,
    grid=(NB,),
    in_specs=[_row_spec, _s16_spec, _one_spec],
    out_specs=[_row_spec, _s16_spec],
    out_shape=[jax.ShapeDtypeStruct((NP, D), jnp.float32),
               jax.ShapeDtypeStruct((NP, SL), jnp.float32)],
)


def _tc_d(p2_ref, den_ref, degs_ref, w2_ref, t3_ref):
    num = p2_ref[0] + p2_ref[1]
    den = den_ref[0] + den_ref[1]
    h_att = jnp.maximum(num / jnp.maximum(den[:, 0:1], 1e-16), 0.0)
    dout = degs_ref[0, 0] + degs_ref[0, 1]
    n = lax.rsqrt(jnp.maximum(dout[:, 0:1], 1.0))
    t3_ref[...] = jnp.dot(h_att, w2_ref[...],
                          preferred_element_type=jnp.float32) * n


_tc_call_d = pl.pallas_call(
    _tc_d,
    grid=(NB,),
    in_specs=[pl.BlockSpec((2, BLK, D), lambda i: (0, i, 0)),
              pl.BlockSpec((2, BLK, SL), lambda i: (0, i, 0)),
              _deg_spec, _w_spec],
    out_specs=_row_spec,
    out_shape=jax.ShapeDtypeStruct((NP, D), jnp.float32),
)


def _tc_e(p3_ref, degs_ref, b2_ref, o_ref):
    din = degs_ref[1, 0] + degs_ref[1, 1]
    n = lax.rsqrt(jnp.maximum(din[:, 0:1], 1.0))
    o_ref[...] = jnp.maximum((p3_ref[0] + p3_ref[1]) * n + b2_ref[...], 0.0)


_tc_call_e = pl.pallas_call(
    _tc_e,
    grid=(NB,),
    in_specs=[pl.BlockSpec((2, BLK, D), lambda i: (0, i, 0)),
              _deg_spec, _vec_spec],
    out_specs=_row_spec,
    out_shape=jax.ShapeDtypeStruct((N, D), jnp.float32),
)


def kernel(x, edge_index, mask, W1, b1, Wl, a, W2, b2):
    f32 = jnp.float32
    src = edge_index[0]
    dst = edge_index[1]
    padn = EP - E
    fill = (N + (jnp.arange(padn, dtype=jnp.int32) % (NP - N))).astype(jnp.int32)
    src_t = jnp.concatenate([src, fill]).reshape(NTILES, NPH, CPP, CHUNK)
    dst_t = jnp.concatenate([dst, fill]).reshape(NTILES, NPH, CPP, CHUNK)
    src8_t = src_t * SL
    x_pad = jnp.concatenate([x, jnp.zeros((NP - N, D), f32)], axis=0)
    zeros1d = jnp.zeros((RPT,), f32)
    zerosrow = jnp.zeros((RPT, D), f32)
    b1r = b1.reshape(1, D)
    b2r = b2.reshape(1, D)
    maskr = mask.reshape(1, D)
    a_row = a.reshape(1, D)

    ones128 = jnp.ones((CHUNK,), f32)
    d2_flat = _deg2_call(src_t, dst_t, zeros1d, ones128)
    degs = jnp.broadcast_to(
        d2_flat.reshape(2, 2, NP).transpose(1, 0, 2)[:, :, :, None],
        (2, 2, NP, SL))
    y1 = _tc_call_a0(x_pad, W1)
    t1 = _tc_call_a1(y1, degs)
    p1 = _seg_call(t1, src_t, dst_t, zerosrow)
    h2, e16, m = _tc_call_b(p1, degs, b1r, maskr, Wl, a_row)
    t2, w16 = _tc_call_c(h2, e16, m)
    p2, den_flat = _segden_call(t2, w16.reshape(NP * SL), src_t, dst_t,
                                src8_t, zerosrow, zeros1d)
    den = jnp.broadcast_to(den_flat.reshape(2, NP, 1), (2, NP, SL))
    t3 = _tc_call_d(p2, den, degs, W2)
    p3 = _seg_call(t3, src_t, dst_t, zerosrow)
    return _tc_call_e(p3, degs, b2r)
